# Initial kernel scaffold; baseline (speedup 1.0000x reference)
import jax
import jax.numpy as jnp
from jax.experimental import pallas as pl


def kernel(x, edge_index, W1, b1, W2, b2):
    raise NotImplementedError("write your pallas kernel here")


# speedup vs baseline: 10.4378x; 10.4378x over previous
import functools

import jax
import jax.numpy as jnp
from jax import lax
from jax.experimental import pallas as pl
from jax.experimental.pallas import tpu as pltpu
from jax.experimental.pallas import tpu_sc as plsc

N = 10000
NPAD = 10240
D_IN = 128
D_HID = 256
W = 128
CHUNK = 128
NPT = NPAD // 16
NBLK = NPT // CHUNK


def _mesh():
    return plsc.VectorSubcoreMesh(core_axis_name="c", subcore_axis_name="s")


def _fill(ref, val):
    v16 = jnp.full((16,), val, jnp.float32)

    def body(i, carry):
        for j in range(W // 16):
            ref[i, pl.ds(j * 16, 16)] = v16
        return carry

    lax.fori_loop(0, CHUNK, body, 0)


def _degree(dst, epad):
    ept = epad // 2 // 16
    nchunks = ept // CHUNK

    @functools.partial(
        pl.kernel,
        mesh=_mesh(),
        out_type=jax.ShapeDtypeStruct((2 * NPAD, W), jnp.float32),
        scratch_types=[
            pltpu.VMEM_SHARED((NPAD, W), jnp.float32),
            pltpu.VMEM((CHUNK, W), jnp.float32),
            pltpu.VMEM((CHUNK,), jnp.int32),
        ],
    )
    def deg_k(dst_hbm, out_hbm, acc_sh, ones_v, idx_v):
        c = lax.axis_index("c")
        s = lax.axis_index("s")
        _fill(ones_v, 0.0)
        for b in range(NBLK):
            pltpu.sync_copy(ones_v, acc_sh.at[pl.ds(s * NPT + b * CHUNK, CHUNK)])
        _fill(ones_v, 1.0)
        plsc.subcore_barrier()
        base = (c * 16 + s) * ept

        def eb(k, carry):
            pltpu.sync_copy(dst_hbm.at[pl.ds(base + k * CHUNK, CHUNK)], idx_v)
            pltpu.sync_copy(ones_v, acc_sh.at[idx_v], add=True)
            return carry

        lax.fori_loop(0, nchunks, eb, 0)
        plsc.subcore_barrier()
        for b in range(NBLK):
            r0 = s * NPT + b * CHUNK
            pltpu.sync_copy(acc_sh.at[pl.ds(r0, CHUNK)],
                            out_hbm.at[pl.ds(c * NPAD + r0, CHUNK)])

    return deg_k(dst)


def _propagate(xs_flat, src, dst, epad, feat_split):
    ept = (epad if feat_split else epad // 2) // 16
    nchunks = ept // CHUNK

    @functools.partial(
        pl.kernel,
        mesh=_mesh(),
        out_type=jax.ShapeDtypeStruct((2 * NPAD, W), jnp.float32),
        scratch_types=[
            pltpu.VMEM_SHARED((NPAD, W), jnp.float32),
            pltpu.VMEM((CHUNK, W), jnp.float32),
            pltpu.VMEM((CHUNK,), jnp.int32),
            pltpu.VMEM((CHUNK,), jnp.int32),
            pltpu.VMEM((CHUNK,), jnp.int32),
            pltpu.SemaphoreType.DMA,
        ],
    )
    def prop_k(xs_hbm, src_hbm, dst_hbm, out_hbm,
               acc_sh, rows_v, sidx_v, didx_v, soff_v, sem):
        c = lax.axis_index("c")
        s = lax.axis_index("s")
        _fill(rows_v, 0.0)
        for b in range(NBLK):
            pltpu.sync_copy(rows_v, acc_sh.at[pl.ds(s * NPT + b * CHUNK, CHUNK)])
        plsc.subcore_barrier()

        if feat_split:
            off = c * NPAD
            base = s * ept
        else:
            off = c * 0
            base = (c * 16 + s) * ept

        def eb(k, carry):
            e0 = base + k * CHUNK
            pltpu.sync_copy(src_hbm.at[pl.ds(e0, CHUNK)], sidx_v)
            pltpu.sync_copy(dst_hbm.at[pl.ds(e0, CHUNK)], didx_v)

            def ob(i, c2):
                soff_v[pl.ds(i * 16, 16)] = sidx_v[pl.ds(i * 16, 16)] + off
                return c2

            lax.fori_loop(0, CHUNK // 16, ob, 0)
            pltpu.async_copy(xs_hbm.at[soff_v], rows_v, sem).wait()
            pltpu.sync_copy(rows_v, acc_sh.at[didx_v], add=True)
            return carry

        lax.fori_loop(0, nchunks, eb, 0)
        plsc.subcore_barrier()
        for b in range(NBLK):
            r0 = s * NPT + b * CHUNK
            pltpu.sync_copy(acc_sh.at[pl.ds(r0, CHUNK)],
                            out_hbm.at[pl.ds(c * NPAD + r0, CHUNK)])

    return prop_k(xs_flat, src, dst)


def _scale(deg_parts, x_pad):
    RB = 2048

    def body(dp_ref, x_ref, xs_ref, dinv_ref):
        deg = dp_ref[0, :, 0:1] + dp_ref[1, :, 0:1] + 1.0
        dinv = lax.rsqrt(deg)
        dinv_ref[...] = dinv
        xs_ref[...] = x_ref[...] * dinv

    return pl.pallas_call(
        body,
        grid=(NPAD // RB,),
        in_specs=[
            pl.BlockSpec((2, RB, W), lambda r: (0, r, 0)),
            pl.BlockSpec((RB, D_IN), lambda r: (r, 0)),
        ],
        out_specs=[
            pl.BlockSpec((RB, D_IN), lambda r: (r, 0)),
            pl.BlockSpec((RB, 1), lambda r: (r, 0)),
        ],
        out_shape=[
            jax.ShapeDtypeStruct((NPAD, D_IN), jnp.float32),
            jax.ShapeDtypeStruct((NPAD, 1), jnp.float32),
        ],
    )(deg_parts, x_pad)


def _layer1(s1, xs, dinv, W1, b1):
    RB = 2048

    def body(s1_ref, xs_ref, dinv_ref, w_ref, b_ref, out_ref):
        dv = dinv_ref[...]
        t = (s1_ref[0] + s1_ref[1] + xs_ref[...]) * dv
        h = jnp.dot(t, w_ref[...], preferred_element_type=jnp.float32)
        h = jnp.maximum(h + b_ref[...], 0.0) * dv
        out_ref[0] = h[:, : D_HID // 2]
        out_ref[1] = h[:, D_HID // 2:]

    return pl.pallas_call(
        body,
        grid=(NPAD // RB,),
        in_specs=[
            pl.BlockSpec((2, RB, D_IN), lambda r: (0, r, 0)),
            pl.BlockSpec((RB, D_IN), lambda r: (r, 0)),
            pl.BlockSpec((RB, 1), lambda r: (r, 0)),
            pl.BlockSpec((D_IN, D_HID), lambda r: (0, 0)),
            pl.BlockSpec((1, D_HID), lambda r: (0, 0)),
        ],
        out_specs=pl.BlockSpec((2, RB, D_HID // 2), lambda r: (0, r, 0)),
        out_shape=jax.ShapeDtypeStruct((2, NPAD, D_HID // 2), jnp.float32),
    )(s1, xs, dinv, W1, b1.reshape(1, D_HID))


def _layer2(s2, hs, dinv, W2, b2):
    RB = 2048

    def body(s2_ref, hs_ref, dinv_ref, w_ref, b_ref, out_ref):
        dv = dinv_ref[...]
        t = jnp.concatenate(
            [(s2_ref[0] + hs_ref[0]) * dv, (s2_ref[1] + hs_ref[1]) * dv],
            axis=1)
        out_ref[...] = (
            jnp.dot(t, w_ref[...], preferred_element_type=jnp.float32)
            + b_ref[...])

    return pl.pallas_call(
        body,
        grid=(NPAD // RB,),
        in_specs=[
            pl.BlockSpec((2, RB, D_HID // 2), lambda r: (0, r, 0)),
            pl.BlockSpec((2, RB, D_HID // 2), lambda r: (0, r, 0)),
            pl.BlockSpec((RB, 1), lambda r: (r, 0)),
            pl.BlockSpec((D_HID, D_HID), lambda r: (0, 0)),
            pl.BlockSpec((1, D_HID), lambda r: (0, 0)),
        ],
        out_specs=pl.BlockSpec((RB, D_HID), lambda r: (r, 0)),
        out_shape=jax.ShapeDtypeStruct((NPAD, D_HID), jnp.float32),
    )(s2, hs, dinv, W2, b2.reshape(1, D_HID))


def kernel(x, edge_index, W1, b1, W2, b2):
    e = edge_index.shape[1]
    step = 32 * CHUNK
    epad = ((e + step - 1) // step) * step
    ei = edge_index.astype(jnp.int32)
    pad = jnp.full((epad - e,), N, jnp.int32)
    src = jnp.concatenate([ei[0], pad])
    dst = jnp.concatenate([ei[1], pad])
    x_pad = jnp.pad(x, ((0, NPAD - N), (0, 0)))

    deg_parts = _degree(dst, epad).reshape(2, NPAD, W)
    xs, dinv = _scale(deg_parts, x_pad)
    s1 = _propagate(xs, src, dst, epad,
                    feat_split=False).reshape(2, NPAD, D_IN)
    hs = _layer1(s1, xs, dinv, W1, b1)
    s2 = _propagate(hs.reshape(2 * NPAD, D_HID // 2), src, dst, epad,
                    feat_split=True).reshape(2, NPAD, D_HID // 2)
    z = _layer2(s2, hs, dinv, W2, b2)
    return z[:N]
